# Initial kernel scaffold; baseline (speedup 1.0000x reference)
import jax
import jax.numpy as jnp
from jax.experimental import pallas as pl


def kernel(x, h, c, edge_index, W_iouf_w, W_iouf_b, U_iouf_w, U_iouf_b):
    raise NotImplementedError("write your pallas kernel here")


# speedup vs baseline: 4.4306x; 4.4306x over previous
import functools

import jax
import jax.numpy as jnp
from jax import lax
from jax.experimental import pallas as pl
from jax.experimental.pallas import tpu as pltpu
from jax.experimental.pallas import tpu_sc as plsc

N = 10000
H = 128
E = 320000

NS = 16
CHUNK = 128
CHUNKS_PER_TILE = 157
TILE_E = CHUNKS_PER_TILE * CHUNK
PADDED_E = NS * TILE_E
ACC_ROWS = 10240
ZROWS = ACC_ROWS // NS
WB = 624
WB_TAIL = N - NS * WB


def _sc_body(h_hbm, c_hbm, src_hbm, dst_hbm, z_hbm, hin_hbm, cin_hbm,
             sidx_v, didx_v, rows_v, acc, sem):
    cid = lax.axis_index("c")
    tid = lax.axis_index("s")

    pltpu.sync_copy(z_hbm, acc.at[pl.ds(tid * ZROWS, ZROWS)])
    plsc.subcore_barrier()

    def run(table_hbm, out_hbm):
        def chunk_body(i, carry):
            base = tid * TILE_E + i * CHUNK
            pltpu.sync_copy(src_hbm.at[pl.ds(base, CHUNK)], sidx_v)
            pltpu.sync_copy(dst_hbm.at[pl.ds(base, CHUNK)], didx_v)
            pltpu.async_copy(table_hbm.at[sidx_v], rows_v, sem).wait()
            pltpu.sync_copy(rows_v, acc.at[didx_v], add=True)
            return carry
        lax.fori_loop(0, CHUNKS_PER_TILE, chunk_body, 0)
        plsc.subcore_barrier()
        pltpu.sync_copy(acc.at[pl.ds(tid * WB, WB)],
                        out_hbm.at[pl.ds(tid * WB, WB)])

        @pl.when(tid == NS - 1)
        def _():
            pltpu.sync_copy(acc.at[pl.ds(NS * WB, WB_TAIL)],
                            out_hbm.at[pl.ds(NS * WB, WB_TAIL)])

    @pl.when(cid == 0)
    def _():
        run(h_hbm, hin_hbm)

    @pl.when(cid == 1)
    def _():
        run(c_hbm, cin_hbm)


_sc_segment_sums = functools.partial(
    pl.kernel,
    out_type=[jax.ShapeDtypeStruct((N, H), jnp.float32),
              jax.ShapeDtypeStruct((N, H), jnp.float32)],
    mesh=plsc.VectorSubcoreMesh(core_axis_name="c", subcore_axis_name="s"),
    scratch_types=[
        pltpu.VMEM((CHUNK,), jnp.int32),
        pltpu.VMEM((CHUNK,), jnp.int32),
        pltpu.VMEM((CHUNK, H), jnp.float32),
        pltpu.VMEM_SHARED((ACC_ROWS, H), jnp.float32),
        pltpu.SemaphoreType.DMA,
    ],
)(_sc_body)


def _lstm_body(x_ref, hin_ref, cin_ref, wt_ref, ut_ref, b_ref,
               hout_ref, cout_ref):
    s = (jnp.dot(x_ref[...], wt_ref[...], preferred_element_type=jnp.float32)
         + jnp.dot(hin_ref[...], ut_ref[...],
                   preferred_element_type=jnp.float32)
         + b_ref[...])
    i = jax.nn.sigmoid(s[:, 0:H])
    o = jax.nn.sigmoid(s[:, H:2 * H])
    u = jnp.tanh(s[:, 2 * H:3 * H])
    f = jax.nn.sigmoid(s[:, 3 * H:4 * H])
    c_new = i * u + f * cin_ref[...]
    cout_ref[...] = c_new
    hout_ref[...] = o * jnp.tanh(c_new)


def _lstm_tc(x, h_in, c_in, wt, ut, b):
    blk = 1000
    grid = (N // blk,)
    return pl.pallas_call(
        _lstm_body,
        grid=grid,
        in_specs=[
            pl.BlockSpec((blk, H), lambda i: (i, 0)),
            pl.BlockSpec((blk, H), lambda i: (i, 0)),
            pl.BlockSpec((blk, H), lambda i: (i, 0)),
            pl.BlockSpec((H, 4 * H), lambda i: (0, 0)),
            pl.BlockSpec((H, 4 * H), lambda i: (0, 0)),
            pl.BlockSpec((1, 4 * H), lambda i: (0, 0)),
        ],
        out_specs=[pl.BlockSpec((blk, H), lambda i: (i, 0)),
                   pl.BlockSpec((blk, H), lambda i: (i, 0))],
        out_shape=[jax.ShapeDtypeStruct((N, H), jnp.float32),
                   jax.ShapeDtypeStruct((N, H), jnp.float32)],
    )(x, h_in, c_in, wt, ut, b)


def kernel(x, h, c, edge_index, W_iouf_w, W_iouf_b, U_iouf_w, U_iouf_b):
    src = edge_index[0]
    dst = edge_index[1]
    pad = PADDED_E - E
    src_p = jnp.concatenate([src, jnp.zeros((pad,), jnp.int32)])
    dst_p = jnp.concatenate([dst, jnp.full((pad,), N, jnp.int32)])
    zeros = jnp.zeros((ZROWS, H), jnp.float32)

    h_in, c_in = _sc_segment_sums(h, c, src_p, dst_p, zeros)

    wt = W_iouf_w.T
    ut = U_iouf_w.T
    b = (W_iouf_b + U_iouf_b).reshape(1, 4 * H)
    return _lstm_tc(x, h_in, c_in, wt, ut, b)
